# baseline (speedup 1.0000x reference)
import functools

import jax
import jax.numpy as jnp
from jax import lax
from jax.experimental import pallas as pl
from jax.experimental.pallas import tpu as pltpu
from jax.experimental.pallas import tpu_sc as plsc

VOCAB = 1000000
EMBED_DIM = 32
BATCH = 16384
FIELDS = 26

NC = 2
NS = 16
LANES = 16
NW = NC * NS

B = BATCH * FIELDS
B_PER_W = BATCH // NW
BCHUNK = 64
N_CHUNKS = B_PER_W // BCHUNK
CHUNK = BCHUNK * FIELDS
KK = BCHUNK // LANES


def _body(table_hbm, idx_hbm, out_hbm, idx_raw, idx_safe, rows, oblk, sem):
    wid = lax.axis_index("s") * NC + lax.axis_index("c")
    bbase = wid * B_PER_W

    iota = lax.iota(jnp.int32, 16)
    iota26 = iota * FIELDS

    @pl.loop(0, N_CHUNKS)
    def _chunk(k):
        b0 = bbase + k * BCHUNK
        off = b0 * FIELDS
        pltpu.sync_copy(idx_hbm.at[pl.ds(off, CHUNK)], idx_raw)

        def _grp(g, minv):
            f = g // KK
            kk = g - f * KK
            v = plsc.load_gather(idx_raw, [iota26 + (kk * LANES * FIELDS + f)])
            idx_safe[pl.ds(f * BCHUNK + kk * LANES, LANES)] = jnp.maximum(v, 1) - 1
            return jnp.minimum(minv, v)

        minv = lax.fori_loop(
            0, FIELDS * KK, _grp, jnp.full((LANES,), VOCAB, jnp.int32)
        )

        pltpu.async_copy(table_hbm.at[idx_safe], rows, sem).wait()

        chunk_min = minv[0]
        for lane in range(1, LANES):
            chunk_min = jnp.minimum(chunk_min, minv[lane])

        @pl.when(chunk_min == 0)
        def _fixup():
            zeros = jnp.zeros((LANES,), jnp.float32)

            def _fix2(g, c):
                f = g // KK
                kk = g - f * KK
                v = plsc.load_gather(
                    idx_raw, [iota26 + (kk * LANES * FIELDS + f)]
                )
                m = v == 0
                row_ids = f * BCHUNK + kk * LANES + iota
                for col in range(EMBED_DIM):
                    plsc.store_scatter(
                        rows,
                        [row_ids, jnp.full((LANES,), col, jnp.int32)],
                        zeros,
                        mask=m,
                    )
                return c

            lax.fori_loop(0, FIELDS * KK, _fix2, jnp.int32(0))

        @pl.loop(0, FIELDS)
        def _shuf(f):
            for d in range(EMBED_DIM):
                for kk in range(KK):
                    rowix = iota + (f * BCHUNK + kk * LANES)
                    v = plsc.load_gather(
                        rows, [rowix, jnp.full((LANES,), d, jnp.int32)]
                    )
                    oblk[f, d, pl.ds(kk * LANES, LANES)] = v

        pltpu.sync_copy(oblk, out_hbm.at[:, :, pl.ds(b0, BCHUNK)])


@functools.cache
def _sc_embed():
    return pl.kernel(
        _body,
        out_type=jax.ShapeDtypeStruct((FIELDS, EMBED_DIM, BATCH), jnp.float32),
        mesh=plsc.VectorSubcoreMesh(
            core_axis_name="c", subcore_axis_name="s", num_cores=NC, num_subcores=NS
        ),
        scratch_types=[
            pltpu.VMEM((CHUNK,), jnp.int32),
            pltpu.VMEM((CHUNK,), jnp.int32),
            pltpu.VMEM((CHUNK, EMBED_DIM), jnp.float32),
            pltpu.VMEM((FIELDS, EMBED_DIM, BCHUNK), jnp.float32),
            pltpu.SemaphoreType.DMA,
        ],
        compiler_params=pltpu.CompilerParams(
            needs_layout_passes=False, use_tc_tiling_on_sc=False
        ),
    )


def kernel(inputs, embedding_matrix):
    idx = inputs.reshape(-1).astype(jnp.int32)
    out_t = _sc_embed()(embedding_matrix, idx)
    return out_t.transpose(2, 0, 1)


# speedup vs baseline: 1.1003x; 1.1003x over previous
import functools

import jax
import jax.numpy as jnp
from jax import lax
from jax.experimental import pallas as pl
from jax.experimental.pallas import tpu as pltpu
from jax.experimental.pallas import tpu_sc as plsc

VOCAB = 1000000
EMBED_DIM = 32
BATCH = 16384
FIELDS = 26

NC = 2
NS = 16
LANES = 16
NW = NC * NS

B = BATCH * FIELDS
B_PER_W = BATCH // NW
BCHUNK = 64
N_CHUNKS = B_PER_W // BCHUNK
CHUNK = BCHUNK * FIELDS
KK = BCHUNK // LANES


def _body(table_hbm, idx_hbm, out_hbm, idx_raw, idx_safe, rows, oblk, sem):
    wid = lax.axis_index("s") * NC + lax.axis_index("c")
    bbase = wid * B_PER_W

    iota = lax.iota(jnp.int32, 16)
    iota26 = iota * FIELDS

    @pl.loop(0, N_CHUNKS)
    def _chunk(k):
        b0 = bbase + k * BCHUNK
        off = b0 * FIELDS
        pltpu.sync_copy(idx_hbm.at[pl.ds(off, CHUNK)], idx_raw)

        def _grp(g, minv):
            f = g // KK
            kk = g - f * KK
            v = plsc.load_gather(idx_raw, [iota26 + (kk * LANES * FIELDS + f)])
            idx_safe[pl.ds(f * BCHUNK + kk * LANES, LANES)] = jnp.maximum(v, 1) - 1
            return jnp.minimum(minv, v)

        minv = lax.fori_loop(
            0, FIELDS * KK, _grp, jnp.full((LANES,), VOCAB, jnp.int32)
        )

        pltpu.async_copy(table_hbm.at[idx_safe], rows, sem).wait()

        chunk_min = minv[0]
        for lane in range(1, LANES):
            chunk_min = jnp.minimum(chunk_min, minv[lane])

        @pl.when(chunk_min == 0)
        def _fixup():
            zeros = jnp.zeros((LANES,), jnp.float32)

            def _fix2(g, c):
                f = g // KK
                kk = g - f * KK
                v = plsc.load_gather(
                    idx_raw, [iota26 + (kk * LANES * FIELDS + f)]
                )
                m = v == 0
                row_ids = f * BCHUNK + kk * LANES + iota
                for col in range(EMBED_DIM):
                    plsc.store_scatter(
                        rows,
                        [row_ids, jnp.full((LANES,), col, jnp.int32)],
                        zeros,
                        mask=m,
                    )
                return c

            lax.fori_loop(0, FIELDS * KK, _fix2, jnp.int32(0))

        @pl.loop(0, FIELDS)
        def _shuf(f):
            base = f * BCHUNK
            rowixs = [iota + (base + kk * LANES) for kk in range(KK)]
            for d0 in range(0, EMBED_DIM, 4):
                vs = []
                for d in range(d0, d0 + 4):
                    cd = jnp.full((LANES,), d, jnp.int32)
                    for kk in range(KK):
                        vs.append(
                            (d, kk, plsc.load_gather(rows, [rowixs[kk], cd]))
                        )
                for d, kk, v in vs:
                    oblk[f, d, pl.ds(kk * LANES, LANES)] = v

        pltpu.sync_copy(oblk, out_hbm.at[:, :, pl.ds(b0, BCHUNK)])


@functools.cache
def _sc_embed():
    return pl.kernel(
        _body,
        out_type=jax.ShapeDtypeStruct((FIELDS, EMBED_DIM, BATCH), jnp.float32),
        mesh=plsc.VectorSubcoreMesh(
            core_axis_name="c", subcore_axis_name="s", num_cores=NC, num_subcores=NS
        ),
        scratch_types=[
            pltpu.VMEM((CHUNK,), jnp.int32),
            pltpu.VMEM((CHUNK,), jnp.int32),
            pltpu.VMEM((CHUNK, EMBED_DIM), jnp.float32),
            pltpu.VMEM((FIELDS, EMBED_DIM, BCHUNK), jnp.float32),
            pltpu.SemaphoreType.DMA,
        ],
        compiler_params=pltpu.CompilerParams(
            needs_layout_passes=False, use_tc_tiling_on_sc=False
        ),
    )


def kernel(inputs, embedding_matrix):
    idx = inputs.reshape(-1).astype(jnp.int32)
    out_t = _sc_embed()(embedding_matrix, idx)
    return out_t.transpose(2, 0, 1)


# speedup vs baseline: 1.4302x; 1.2998x over previous
import functools

import jax
import jax.numpy as jnp
from jax import lax
from jax.experimental import pallas as pl
from jax.experimental.pallas import tpu as pltpu
from jax.experimental.pallas import tpu_sc as plsc

VOCAB = 1000000
EMBED_DIM = 32
BATCH = 16384
FIELDS = 26

NC = 2
NS = 16
LANES = 16
NW = NC * NS

B = BATCH * FIELDS
B_PER_W = BATCH // NW
BCHUNK = 64
N_CHUNKS = B_PER_W // BCHUNK
CHUNK = BCHUNK * FIELDS
KK = BCHUNK // LANES


def _body(table_hbm, idx_hbm, out_hbm, idx_raw, idx_safe, rows, oblk, sem):
    wid = lax.axis_index("s") * NC + lax.axis_index("c")
    bbase = wid * B_PER_W

    iota = lax.iota(jnp.int32, 16)
    iota26 = iota * FIELDS

    @pl.loop(0, N_CHUNKS)
    def _chunk(k):
        b0 = bbase + k * BCHUNK
        off = b0 * FIELDS
        pltpu.sync_copy(idx_hbm.at[pl.ds(off, CHUNK)], idx_raw)

        def _grp(g, minv):
            f = g // KK
            kk = g - f * KK
            v = plsc.load_gather(idx_raw, [iota26 + (kk * LANES * FIELDS + f)])
            idx_safe[pl.ds(f * BCHUNK + kk * LANES, LANES)] = jnp.maximum(v, 1) - 1
            return jnp.minimum(minv, v)

        minv = lax.fori_loop(
            0, FIELDS * KK, _grp, jnp.full((LANES,), VOCAB, jnp.int32)
        )

        pltpu.async_copy(table_hbm.at[idx_safe], rows, sem).wait()

        chunk_min = minv[0]
        for lane in range(1, LANES):
            chunk_min = jnp.minimum(chunk_min, minv[lane])

        @pl.when(chunk_min == 0)
        def _fixup():
            zeros = jnp.zeros((LANES,), jnp.float32)

            def _fix2(g, c):
                f = g // KK
                kk = g - f * KK
                v = plsc.load_gather(
                    idx_raw, [iota26 + (kk * LANES * FIELDS + f)]
                )
                m = v == 0
                row_ids = f * BCHUNK + kk * LANES + iota
                for col in range(EMBED_DIM):
                    plsc.store_scatter(
                        rows,
                        [row_ids, jnp.full((LANES,), col, jnp.int32)],
                        zeros,
                        mask=m,
                    )
                return c

            lax.fori_loop(0, FIELDS * KK, _fix2, jnp.int32(0))

        @pl.loop(0, FIELDS)
        def _shuf(f):
            base = f * BCHUNK
            rowv = [iota + (f * EMBED_DIM + h * LANES) for h in range(2)]
            for bb0 in range(0, BCHUNK, 8):
                vs = []
                for bb in range(bb0, bb0 + 8):
                    for h in range(2):
                        vs.append(
                            (bb, h, rows[base + bb, pl.ds(h * LANES, LANES)])
                        )
                for bb, h, v in vs:
                    plsc.store_scatter(
                        oblk, [rowv[h], jnp.full((LANES,), bb, jnp.int32)], v
                    )

        pltpu.sync_copy(
            oblk.at[:, pl.ds(0, BCHUNK)], out_hbm.at[:, pl.ds(b0, BCHUNK)]
        )


@functools.cache
def _sc_embed():
    return pl.kernel(
        _body,
        out_type=jax.ShapeDtypeStruct((FIELDS * EMBED_DIM, BATCH), jnp.float32),
        mesh=plsc.VectorSubcoreMesh(
            core_axis_name="c", subcore_axis_name="s", num_cores=NC, num_subcores=NS
        ),
        scratch_types=[
            pltpu.VMEM((CHUNK,), jnp.int32),
            pltpu.VMEM((CHUNK,), jnp.int32),
            pltpu.VMEM((CHUNK, EMBED_DIM), jnp.float32),
            pltpu.VMEM((FIELDS * EMBED_DIM, BCHUNK + 1), jnp.float32),
            pltpu.SemaphoreType.DMA,
        ],
        compiler_params=pltpu.CompilerParams(
            needs_layout_passes=False, use_tc_tiling_on_sc=False
        ),
    )


def kernel(inputs, embedding_matrix):
    idx = inputs.reshape(-1).astype(jnp.int32)
    out_t = _sc_embed()(embedding_matrix, idx)
    return out_t.reshape(FIELDS, EMBED_DIM, BATCH).transpose(2, 0, 1)
